# baseline (speedup 1.0000x reference)
import jax
import jax.numpy as jnp
from jax import lax
from jax.experimental import pallas as pl
from jax.experimental.pallas import tpu as pltpu
from jax.experimental.pallas import tpu_sc as plsc

N_CORES = 2
N_SUBCORES = 16
LANES = 16
N_WORKERS = N_CORES * N_SUBCORES

BATCH = 16384
DIM = 32
B_PER_W = BATCH // N_WORKERS
WBLK = 128
NBLK = (1000000 + WBLK - 1) // WBLK
OWN = (NBLK + N_WORKERS - 1) // N_WORKERS
NR2 = 6
LEAD = 4
EPAD = BATCH + LANES


def _scalar(ref, k):
    return ref[pl.ds(k, LANES)][0]


def _gather_one(idx_hbm, twt_hbm, oe_hbm, idx_all, mr, mp, br, bp, pres,
                win, stage, wsem, ssem, lo, lane):
    ones = jnp.ones((LANES,), jnp.int32)
    zeros16 = jnp.zeros((LANES,), jnp.int32)

    for z in range((OWN + 2 * LANES - 1) // LANES):
        pres[pl.ds(z * LANES, LANES)] = zeros16

    pltpu.sync_copy(idx_hbm, idx_all.at[pl.ds(0, BATCH)])

    def p1(g, off):
        rv = idx_all[pl.ds(g * LANES, LANES)]
        rel = lax.shift_right_logical(rv, 7) - lo
        m = (rel >= 0) & (rel < OWN)
        plsc.store_scatter(pres, [rel], ones, mask=m)
        pos = g * LANES + lane
        plsc.store_compressed(mr.at[pl.ds(off, LANES)], rv, mask=m)
        plsc.store_compressed(mp.at[pl.ds(off, LANES)], pos, mask=m)
        return off + plsc.all_reduce_population_count(m)[0]

    n = lax.fori_loop(0, BATCH // LANES, p1, 0)
    nv = lax.div(n + LANES - 1, LANES)

    def fire(b):
        blk = lo + b
        c0 = pl.multiple_of(blk * WBLK, WBLK)
        slot = lax.rem(b, NR2)
        pltpu.make_async_copy(
            twt_hbm.at[:, pl.ds(c0, WBLK)],
            win.at[pl.ds(slot * DIM, DIM), :],
            wsem.at[slot],
        ).start()

    def wait(b):
        blk = lo + b
        c0 = pl.multiple_of(blk * WBLK, WBLK)
        slot = lax.rem(b, NR2)
        pltpu.make_async_copy(
            twt_hbm.at[:, pl.ds(c0, WBLK)],
            win.at[pl.ds(slot * DIM, DIM), :],
            wsem.at[slot],
        ).wait()

    for s in range(LEAD):
        @pl.when(_scalar(pres, s) > 0)
        def _():
            fire(s)

    def blk_step(b, carry):
        blk = lo + b
        slot = lax.rem(b, NR2)

        @pl.when(_scalar(pres, b) > 0)
        def _():
            def mc(q, cnt):
                rv = mr[pl.ds(q * LANES, LANES)]
                pv = mp[pl.ds(q * LANES, LANES)]
                m = (lax.shift_right_logical(rv, 7) == blk) & (
                    q * LANES + lane < n)
                plsc.store_compressed(br.at[pl.ds(cnt, LANES)], rv, mask=m)
                plsc.store_compressed(bp.at[pl.ds(cnt, LANES)], pv, mask=m)
                return cnt + plsc.all_reduce_population_count(m)[0]

            cnt = lax.fori_loop(0, nv, mc, 0)
            wait(b)

            def eg(q, carry2):
                rv = br[pl.ds(q * LANES, LANES)]
                pv = bp[pl.ds(q * LANES, LANES)]
                valid = q * LANES + lane < cnt
                pvs = jnp.where(valid, pv, BATCH + lane)
                cu = lax.bitwise_and(rv, WBLK - 1)
                for d in range(DIM):
                    dd = lax.bitwise_and(lane + d, DIM - 1)
                    v = plsc.load_gather(win, [slot * DIM + dd, cu])
                    plsc.store_scatter(stage, [lane, dd], v)
                pltpu.async_copy(stage, oe_hbm.at[pvs], ssem).wait()
                return carry2

            lax.fori_loop(0, lax.div(cnt + LANES - 1, LANES), eg, 0)

        @pl.when((b + LEAD < OWN) & (_scalar(pres, b + LEAD) > 0))
        def _():
            fire(b + LEAD)

        return carry

    lax.fori_loop(0, OWN, blk_step, 0)


def _mf_gather(user_hbm, item_hbm, uwt_hbm, iwt_hbm, ue_hbm, ie_hbm,
               idx_all, mr, mp, br, bp, pres, win, stage, wsem, ssem):
    wid = lax.axis_index("s") * N_CORES + lax.axis_index("c")
    lo = wid * OWN
    lane = lax.iota(jnp.int32, LANES)
    _gather_one(user_hbm, uwt_hbm, ue_hbm, idx_all, mr, mp, br, bp, pres,
                win, stage, wsem, ssem, lo, lane)
    _gather_one(item_hbm, iwt_hbm, ie_hbm, idx_all, mr, mp, br, bp, pres,
                win, stage, wsem, ssem, lo, lane)


def _mf_dot(ue_hbm, ie_hbm, out_hbm, ubuf, ibuf, outv):
    wid = lax.axis_index("s") * N_CORES + lax.axis_index("c")
    base = wid * B_PER_W
    lane = lax.iota(jnp.int32, LANES)
    CH = 128

    for c in range(B_PER_W // CH):
        pltpu.sync_copy(ue_hbm.at[pl.ds(base + c * CH, CH), :], ubuf)
        pltpu.sync_copy(ie_hbm.at[pl.ds(base + c * CH, CH), :], ibuf)

        def group(g, carry):
            row = g * LANES + lane
            acc = jnp.zeros((LANES,), jnp.float32)
            for d in range(DIM):
                dd = lax.bitwise_and(lane + d, DIM - 1)
                u = plsc.load_gather(ubuf, [row, dd])
                v = plsc.load_gather(ibuf, [row, dd])
                acc = acc + u * v
            outv[pl.ds(g * LANES, LANES)] = 1.0 / (1.0 + jnp.exp(-acc))
            return carry

        lax.fori_loop(0, CH // LANES, group, 0)
        pltpu.sync_copy(outv, out_hbm.at[pl.ds(base + c * CH, CH)])


def kernel(user, item, user_emb_weight, item_emb_weight):
    uwt = user_emb_weight.T
    iwt = item_emb_weight.T
    mesh = plsc.VectorSubcoreMesh(core_axis_name="c", subcore_axis_name="s")
    cp = pltpu.CompilerParams(
        needs_layout_passes=False, use_tc_tiling_on_sc=True)

    ga = pl.kernel(
        _mf_gather,
        out_type=[
            jax.ShapeDtypeStruct((EPAD, WBLK), jnp.float32),
            jax.ShapeDtypeStruct((EPAD, WBLK), jnp.float32),
        ],
        mesh=mesh,
        compiler_params=cp,
        scratch_types=[
            pltpu.VMEM((BATCH + LANES,), jnp.int32),
            pltpu.VMEM((BATCH + LANES,), jnp.int32),
            pltpu.VMEM((BATCH + LANES,), jnp.int32),
            pltpu.VMEM((BATCH + LANES,), jnp.int32),
            pltpu.VMEM((BATCH + LANES,), jnp.int32),
            pltpu.VMEM((OWN + 2 * LANES,), jnp.int32),
            pltpu.VMEM((NR2 * DIM, WBLK), jnp.float32),
            pltpu.VMEM((LANES, WBLK), jnp.float32),
            pltpu.SemaphoreType.DMA((NR2,)),
            pltpu.SemaphoreType.DMA,
        ],
    )
    ue, ie = ga(user, item, uwt, iwt)

    dot = pl.kernel(
        _mf_dot,
        out_type=jax.ShapeDtypeStruct((BATCH,), jnp.float32),
        mesh=mesh,
        compiler_params=cp,
        scratch_types=[
            pltpu.VMEM((128, WBLK), jnp.float32),
            pltpu.VMEM((128, WBLK), jnp.float32),
            pltpu.VMEM((128,), jnp.float32),
        ],
    )
    return dot(ue, ie)


# speedup vs baseline: 3.8646x; 3.8646x over previous
import jax
import jax.numpy as jnp
from jax import lax
from jax.experimental import pallas as pl
from jax.experimental.pallas import tpu as pltpu
from jax.experimental.pallas import tpu_sc as plsc

N_CORES = 2
N_SUBCORES = 16
LANES = 16
N_WORKERS = N_CORES * N_SUBCORES

BATCH = 16384
DIM = 32
B_PER_W = BATCH // N_WORKERS
WBLK = 128
NRING = 10
SLACK = 3


def _win_copy(tbl_hbm, win, sem, ridx, slot):
    c0 = pl.multiple_of(lax.bitwise_and(ridx, jnp.int32(-WBLK)), WBLK)
    return pltpu.make_async_copy(
        tbl_hbm.at[:, pl.ds(c0, WBLK)],
        win.at[pl.ds(slot * DIM, DIM), :],
        sem.at[slot],
    )


def _idx_at(idx_ref, k):
    return idx_ref[pl.ds(k, LANES)][0]


def _mf_body(user_hbm, item_hbm, uwt_hbm, iwt_hbm, out_hbm,
             uidx, iidx, uwin, iwin, urows, irows, outv, usem, isem):
    wid = lax.axis_index("s") * N_CORES + lax.axis_index("c")
    base = wid * B_PER_W

    pltpu.sync_copy(user_hbm.at[pl.ds(base, B_PER_W)],
                    uidx.at[pl.ds(0, B_PER_W)])
    pltpu.sync_copy(item_hbm.at[pl.ds(base, B_PER_W)],
                    iidx.at[pl.ds(0, B_PER_W)])

    lane = lax.iota(jnp.int32, LANES)

    for s in range(NRING - SLACK):
        _win_copy(uwt_hbm, uwin, usem, _idx_at(uidx, s), s).start()
        _win_copy(iwt_hbm, iwin, isem, _idx_at(iidx, s), s).start()

    def extract(win, slot, col):
        row_lo = slot * DIM + lane
        cvec = jnp.full((LANES,), col, jnp.int32) + lane * 0
        lo = plsc.load_gather(win, [row_lo, cvec])
        hi = plsc.load_gather(win, [row_lo + LANES, cvec])
        return lo, hi

    def gather_step(k, carry):
        slot = lax.rem(k, NRING)
        ur = _idx_at(uidx, k)
        ir = _idx_at(iidx, k)
        _win_copy(uwt_hbm, uwin, usem, ur, slot).wait()
        _win_copy(iwt_hbm, iwin, isem, ir, slot).wait()
        ulo, uhi = extract(uwin, slot, lax.bitwise_and(ur, WBLK - 1))
        ilo, ihi = extract(iwin, slot, lax.bitwise_and(ir, WBLK - 1))
        urows[pl.ds(k * DIM, LANES)] = ulo
        urows[pl.ds(k * DIM + LANES, LANES)] = uhi
        irows[pl.ds(k * DIM, LANES)] = ilo
        irows[pl.ds(k * DIM + LANES, LANES)] = ihi

        nxt = k + NRING - SLACK
        nslot = lax.rem(nxt, NRING)

        @pl.when(nxt < B_PER_W)
        def _():
            _win_copy(uwt_hbm, uwin, usem, _idx_at(uidx, nxt), nslot).start()
            _win_copy(iwt_hbm, iwin, isem, _idx_at(iidx, nxt), nslot).start()

        return carry

    lax.fori_loop(0, B_PER_W, gather_step, 0)

    def group(g, carry):
        rbase = g * LANES * DIM + lane * DIM
        acc = jnp.zeros((LANES,), jnp.float32)
        for d in range(DIM):
            off = rbase + lax.bitwise_and(lane + d, DIM - 1)
            u = plsc.load_gather(urows, [off])
            v = plsc.load_gather(irows, [off])
            acc = acc + u * v
        outv[pl.ds(g * LANES, LANES)] = 1.0 / (1.0 + jnp.exp(-acc))
        return carry

    lax.fori_loop(0, B_PER_W // LANES, group, 0)
    pltpu.sync_copy(outv, out_hbm.at[pl.ds(base, B_PER_W)])


def kernel(user, item, user_emb_weight, item_emb_weight):
    uwt = user_emb_weight.T
    iwt = item_emb_weight.T
    mesh = plsc.VectorSubcoreMesh(core_axis_name="c", subcore_axis_name="s")
    f = pl.kernel(
        _mf_body,
        out_type=jax.ShapeDtypeStruct((BATCH,), jnp.float32),
        mesh=mesh,
        compiler_params=pltpu.CompilerParams(
            needs_layout_passes=False, use_tc_tiling_on_sc=True),
        scratch_types=[
            pltpu.VMEM((B_PER_W + LANES,), jnp.int32),
            pltpu.VMEM((B_PER_W + LANES,), jnp.int32),
            pltpu.VMEM((NRING * DIM, WBLK), jnp.float32),
            pltpu.VMEM((NRING * DIM, WBLK), jnp.float32),
            pltpu.VMEM((B_PER_W * DIM,), jnp.float32),
            pltpu.VMEM((B_PER_W * DIM,), jnp.float32),
            pltpu.VMEM((B_PER_W,), jnp.float32),
            pltpu.SemaphoreType.DMA((NRING,)),
            pltpu.SemaphoreType.DMA((NRING,)),
        ],
    )
    return f(user, item, uwt, iwt)
